# baseline (speedup 1.0000x reference)
import functools

import numpy as np

import jax
import jax.numpy as jnp
from jax import lax
from jax.experimental import pallas as pl
from jax.experimental.pallas import tpu as pltpu
from jax.experimental.pallas import tpu_sc as plsc

_K = 16
_SCALE = (1.0 + 1e-06) ** -0.5
_NW = 32


def _leaky(x):
    return jnp.where(x >= 0, x, 0.2 * x)


def _mm(x, w):
    return lax.dot_general(x, w, (((x.ndim - 1,), (0,)), ((), ())),
                           preferred_element_type=jnp.float32)


def _round_up(x, m):
    return (x + m - 1) // m * m


def _largest_div(n, cap, mult=1):
    best = mult
    d = mult
    while d <= min(n, cap):
        if n % d == 0:
            best = d
        d += mult
    return best


def _seg_mats(nb, k):
    psel = np.kron(np.eye(nb, dtype=np.float32), np.ones((1, k), np.float32))
    return psel, psel.T.copy()


def _blk(n, row_bytes, budget=4 * 1024 * 1024):
    p2 = n & (-n)
    cap = max(1, budget // max(row_bytes, 1))
    nb = 1
    while nb * 2 <= p2 and nb * 2 <= cap:
        nb *= 2
    return nb


def _gather_rows(table, idx):
    n_tab, d = table.shape
    ow = _round_up(d, 128)
    b = idx.shape[0]
    bp = _round_up(b, 8 * _NW)
    if bp > b:
        idx = jnp.concatenate([idx, jnp.zeros((bp - b,), jnp.int32)])
    rows_per_w = bp // _NW
    r = _largest_div(rows_per_w, cap=128, mult=8)
    n_ops = rows_per_w // r
    f_cap = max(1, min(8, (448 * 1024) // (r * d * 4)))
    f = _largest_div(n_ops, cap=f_cap)
    g_cnt = n_ops // f
    idx2 = idx.reshape(bp // r, r)

    @functools.partial(
        pl.kernel,
        mesh=plsc.VectorSubcoreMesh(core_axis_name="c", subcore_axis_name="s"),
        compiler_params=pltpu.CompilerParams(use_tc_tiling_on_sc=False),
        out_type=jax.ShapeDtypeStruct((bp, ow), jnp.float32),
        scratch_types=[
            pltpu.VMEM((f, r), jnp.int32),
            pltpu.VMEM((f * r, d), jnp.float32),
            pltpu.SemaphoreType.DMA,
        ],
    )
    def gk(table_hbm, idx_hbm, out_hbm, idx_v, rows_v, sem):
        wid = lax.axis_index("s") * 2 + lax.axis_index("c")
        op0 = wid * n_ops

        def one_group(g):
            gbase = op0 + g * f
            pltpu.sync_copy(idx_hbm.at[pl.ds(gbase, f)], idx_v)
            handles = []
            for j in range(f):
                handles.append(pltpu.async_copy(
                    table_hbm.at[idx_v.at[j]],
                    rows_v.at[pl.ds(j * r, r)], sem))
            for h in handles:
                h.wait()
            if ow == d:
                pltpu.sync_copy(rows_v, out_hbm.at[pl.ds(gbase * r, f * r)])
            else:
                pltpu.sync_copy(
                    rows_v,
                    out_hbm.at[pl.ds(gbase * r, f * r), pl.ds(0, d)])

        if g_cnt == 1:
            one_group(0)
        else:
            def body(g, carry):
                one_group(g)
                return carry
            lax.fori_loop(0, g_cnt, body, 0)

    out = gk(table, idx2)
    return out[:b] if bp > b else out


def _full_spec(shape):
    nd = len(shape)
    return pl.BlockSpec(shape, lambda i, _nd=nd: (0,) * _nd)


def _fc0_mlp1(feats, xyz, w0, b0, w1, b1, w1_out):
    n = feats.shape[0]
    d_f = w0.shape[1]
    d2 = w1.shape[1]
    nb = _blk(n, 128 * 4 * 2)

    def body(x_ref, xyz_ref, w0_ref, b0_ref, w1_ref, b1_ref, feat_ref, t1_ref):
        x = x_ref[...]
        ft = _leaky(_SCALE * (_mm(x, w0_ref[...]) + b0_ref[...]))
        fp = _leaky(_SCALE * (_mm(ft, w1_ref[...]) + b1_ref[...]))
        feat_ref[...] = ft
        pad = w1_out - d2 - 3
        t1_ref[...] = jnp.concatenate(
            [fp, xyz_ref[...], jnp.zeros((fp.shape[0], pad), jnp.float32)],
            axis=1)

    return pl.pallas_call(
        body,
        grid=(n // nb,),
        in_specs=[
            pl.BlockSpec((nb, 3), lambda i: (i, 0)),
            pl.BlockSpec((nb, 3), lambda i: (i, 0)),
            _full_spec(w0.shape), _full_spec((1, d_f)),
            _full_spec(w1.shape), _full_spec((1, d2)),
        ],
        out_specs=[
            pl.BlockSpec((nb, d_f), lambda i: (i, 0)),
            pl.BlockSpec((nb, w1_out), lambda i: (i, 0)),
        ],
        out_shape=[
            jax.ShapeDtypeStruct((n, d_f), jnp.float32),
            jax.ShapeDtypeStruct((n, w1_out), jnp.float32),
        ],
    )(feats, xyz, w0, b0[None, :], w1, b1[None, :])


def _block_a(rows1, xyz, wl1, bl1, wfc, bfc, wm, bm, wl2, bl2, w2_out, w1):
    ow1 = rows1.shape[1]
    n = rows1.shape[0] // _K
    k = _K
    d2 = wl1.shape[1]
    c = 2 * d2

    w_rel, w_tile, w_nx = wl1[1:4], wl1[4:7], wl1[7:10]
    w_comb = jnp.zeros((w1, c), jnp.float32)
    w_comb = w_comb.at[:d2, :d2].set(jnp.eye(d2, dtype=jnp.float32))
    w_comb = w_comb.at[d2:d2 + 3, d2:].set(_SCALE * (w_nx - w_rel))
    wt_pad = jnp.zeros((3, c), jnp.float32).at[:, d2:].set(
        _SCALE * (w_rel + w_tile))
    w0_pad = jnp.zeros((1, c), jnp.float32).at[0, d2:].set(_SCALE * wl1[0])
    b_pad = jnp.zeros((1, c), jnp.float32).at[0, d2:].set(_SCALE * bl1)
    wl2_pad = jnp.zeros((c, d2), jnp.float32).at[d2:].set(wl2)

    nb = _blk(n, 3 * k * 128 * 4)
    psel, bexp = _seg_mats(nb, k)
    ones31 = np.ones((3, 1), np.float32)

    def body(rows_ref, xyz_ref, psel_ref, bexp_ref, ones_ref, wc_ref, wt_ref,
             w0_ref, bp_ref, wfc_ref, bfc_ref, wm_ref, bm_ref, wl2_ref,
             bl2_ref, t2_ref, fx2_ref):
        rows = rows_ref[...][:, :w1]
        xyz_b = xyz_ref[...]
        f0 = _mm(rows, wc_ref[...])
        if c >= 128:
            txyz = _mm(bexp_ref[...], xyz_b)
            rel = txyz - rows[:, d2:d2 + 3]
            dis = jnp.sqrt(_mm(rel * rel, ones_ref[...]) + 1e-12)
            f3 = (f0 + _mm(txyz, wt_ref[...]) + _mm(dis, w0_ref[...])
                  + bp_ref[...])
            lane = lax.broadcasted_iota(jnp.int32, (nb * k, c), 1)
            f = jnp.where(lane < d2, f3, _leaky(f3))
            e = jnp.exp(_mm(f, wfc_ref[...]) + bfc_ref[...])
            den = _mm(psel_ref[...], e)
            num = _mm(psel_ref[...], f * e)
        else:
            r3 = rows.reshape(nb, k, w1)
            rel0 = xyz_b[:, None, 0:1] - r3[:, :, d2:d2 + 1]
            rel1 = xyz_b[:, None, 1:2] - r3[:, :, d2 + 1:d2 + 2]
            rel2 = xyz_b[:, None, 2:3] - r3[:, :, d2 + 2:d2 + 3]
            dis = jnp.sqrt(rel0 * rel0 + rel1 * rel1 + rel2 * rel2 + 1e-12)
            tp = _mm(xyz_b, wt_ref[...])
            f3 = (f0.reshape(nb, k, c) + tp[:, None, :]
                  + dis * w0_ref[...][None] + bp_ref[...][None])
            lane = lax.broadcasted_iota(jnp.int32, (nb, k, c), 2)
            f3d = jnp.where(lane < d2, f3, _leaky(f3))
            f = f3d.reshape(nb * k, c)
            e = jnp.exp(_mm(f, wfc_ref[...]) + bfc_ref[...]).reshape(nb, k, c)
            den = jnp.sum(e, axis=1)
            num = jnp.sum(f3d * e, axis=1)
        agg = num / den
        fagg = _leaky(_SCALE * (_mm(agg, wm_ref[...]) + bm_ref[...]))
        fx2 = _leaky(_SCALE * (_mm(f, wl2_ref[...]) + bl2_ref[...]))
        pad = w2_out - d2
        if pad:
            fagg = jnp.concatenate(
                [fagg, jnp.zeros((nb, pad), jnp.float32)], axis=1)
        t2_ref[...] = fagg
        fx2_ref[...] = fx2

    return pl.pallas_call(
        body,
        grid=(n // nb,),
        in_specs=[
            pl.BlockSpec((nb * k, ow1), lambda i: (i, 0)),
            pl.BlockSpec((nb, 3), lambda i: (i, 0)),
            _full_spec((nb, nb * k)), _full_spec((nb * k, nb)),
            _full_spec((3, 1)),
            _full_spec((w1, c)), _full_spec((3, c)),
            _full_spec((1, c)), _full_spec((1, c)),
            _full_spec(wfc.shape), _full_spec((1, c)),
            _full_spec(wm.shape), _full_spec((1, d2)),
            _full_spec((c, d2)), _full_spec((1, d2)),
        ],
        out_specs=[
            pl.BlockSpec((nb, w2_out), lambda i: (i, 0)),
            pl.BlockSpec((nb * k, d2), lambda i: (i, 0)),
        ],
        out_shape=[
            jax.ShapeDtypeStruct((n, w2_out), jnp.float32),
            jax.ShapeDtypeStruct((n * k, d2), jnp.float32),
        ],
    )(rows1, xyz, psel, bexp, ones31, w_comb, wt_pad, w0_pad, b_pad, wfc,
      bfc[None, :], wm, bm[None, :], wl2_pad, bl2[None, :])


def _block_b(rows2, fx2, feat, wfc, bfc, wm, bm, wmlp2, bmlp2, wsh, bsh):
    ow2 = rows2.shape[1]
    k = _K
    n = rows2.shape[0] // k
    d2 = fx2.shape[1]
    c = 2 * d2
    c2 = wmlp2.shape[1]
    d_in = feat.shape[1]
    nb = _blk(n, 3 * k * 128 * 4)
    psel, _ = _seg_mats(nb, k)

    def body(rows_ref, fx2_ref, feat_ref, psel_ref, wfc_ref, bfc_ref,
             wm_ref, bm_ref, wmlp2_ref, bmlp2_ref, wsh_ref, bsh_ref, out_ref):
        f_n = rows_ref[...][:, :d2]
        fxyz = fx2_ref[...]
        f = jnp.concatenate([f_n, fxyz], axis=1)
        e = jnp.exp(_mm(f, wfc_ref[...]) + bfc_ref[...])
        if c >= 128:
            den = _mm(psel_ref[...], e)
            num = _mm(psel_ref[...], f * e)
        else:
            den = jnp.sum(e.reshape(nb, k, c), axis=1)
            num = jnp.sum((f * e).reshape(nb, k, c), axis=1)
        agg = num / den
        fagg = _leaky(_SCALE * (_mm(agg, wm_ref[...]) + bm_ref[...]))
        fpc = _SCALE * (_mm(fagg, wmlp2_ref[...]) + bmlp2_ref[...])
        sc = _SCALE * (_mm(feat_ref[...], wsh_ref[...]) + bsh_ref[...])
        out_ref[...] = _leaky(fpc + sc)

    return pl.pallas_call(
        body,
        grid=(n // nb,),
        in_specs=[
            pl.BlockSpec((nb * k, ow2), lambda i: (i, 0)),
            pl.BlockSpec((nb * k, d2), lambda i: (i, 0)),
            pl.BlockSpec((nb, d_in), lambda i: (i, 0)),
            _full_spec((nb, nb * k)),
            _full_spec(wfc.shape), _full_spec((1, c)),
            _full_spec(wm.shape), _full_spec((1, c)),
            _full_spec(wmlp2.shape), _full_spec((1, c2)),
            _full_spec(wsh.shape), _full_spec((1, c2)),
        ],
        out_specs=pl.BlockSpec((nb, c2), lambda i: (i, 0)),
        out_shape=jax.ShapeDtypeStruct((n, c2), jnp.float32),
    )(rows2, fx2, feat, psel, wfc, bfc[None, :], wm, bm[None, :], wmlp2,
      bmlp2[None, :], wsh, bsh[None, :])


def _pool_mlp(prows, xyz, w, b, w1_out):
    ow = prows.shape[1]
    c = w.shape[0]
    k = _K
    n = prows.shape[0] // k
    d_o = w.shape[1]
    nb = _blk(n, (k + 2) * max(c, 128) * 4)
    has_xyz = xyz is not None

    def body(*refs):
        if has_xyz:
            (rows_ref, xyz_ref, w_ref, b_ref, fs_ref, t1_ref) = refs
        else:
            (rows_ref, w_ref, b_ref, fs_ref, t1_ref) = refs
        fs = jnp.max(rows_ref[...][:, :c].reshape(nb, k, c), axis=1)
        fp = _leaky(_SCALE * (_mm(fs, w_ref[...]) + b_ref[...]))
        fs_ref[...] = fs
        if has_xyz:
            pad = w1_out - d_o - 3
            fp = jnp.concatenate(
                [fp, xyz_ref[...], jnp.zeros((nb, pad), jnp.float32)], axis=1)
        t1_ref[...] = fp

    in_specs = [pl.BlockSpec((nb * k, ow), lambda i: (i, 0))]
    args = [prows]
    if has_xyz:
        in_specs.append(pl.BlockSpec((nb, 3), lambda i: (i, 0)))
        args.append(xyz)
    in_specs += [_full_spec(w.shape), _full_spec((1, d_o))]
    args += [w, b[None, :]]

    return pl.pallas_call(
        body,
        grid=(n // nb,),
        in_specs=in_specs,
        out_specs=[
            pl.BlockSpec((nb, c), lambda i: (i, 0)),
            pl.BlockSpec((nb, w1_out), lambda i: (i, 0)),
        ],
        out_shape=[
            jax.ShapeDtypeStruct((n, c), jnp.float32),
            jax.ShapeDtypeStruct((n, w1_out), jnp.float32),
        ],
    )(*args)


def _dual_mm(skip, rows, wt, wb, b):
    n, ds = skip.shape
    ow = rows.shape[1]
    df = wb.shape[0]
    d_o = wt.shape[1]
    nb = _blk(n, (ds + ow + d_o) * 4)

    def body(s_ref, r_ref, wt_ref, wb_ref, b_ref, o_ref):
        y = _mm(s_ref[...], wt_ref[...]) + _mm(r_ref[...][:, :df], wb_ref[...])
        o_ref[...] = _leaky(_SCALE * (y + b_ref[...]))

    return pl.pallas_call(
        body,
        grid=(n // nb,),
        in_specs=[
            pl.BlockSpec((nb, ds), lambda i: (i, 0)),
            pl.BlockSpec((nb, ow), lambda i: (i, 0)),
            _full_spec(wt.shape), _full_spec(wb.shape), _full_spec((1, d_o)),
        ],
        out_specs=pl.BlockSpec((nb, d_o), lambda i: (i, 0)),
        out_shape=jax.ShapeDtypeStruct((n, d_o), jnp.float32),
    )(skip, rows, wt, wb, b[None, :])


def _final(skip, rows, wt, wb, bd, w1, b1, w2, b2, w3, b3):
    n, ds = skip.shape
    ow = rows.shape[1]
    df = wb.shape[0]
    ncls = w3.shape[1]
    nb = _blk(n, (ds + ow + 64 + 128) * 4)

    def body(s_ref, r_ref, wt_ref, wb_ref, bd_ref, w1_ref, b1_ref,
             w2_ref, b2_ref, w3_ref, b3_ref, o_ref):
        y = _mm(s_ref[...], wt_ref[...]) + _mm(r_ref[...][:, :df], wb_ref[...])
        y = _leaky(_SCALE * (y + bd_ref[...]))
        y = _leaky(_SCALE * (_mm(y, w1_ref[...]) + b1_ref[...]))
        y = _leaky(_SCALE * (_mm(y, w2_ref[...]) + b2_ref[...]))
        o_ref[...] = _mm(y, w3_ref[...]) + b3_ref[...]

    return pl.pallas_call(
        body,
        grid=(n // nb,),
        in_specs=[
            pl.BlockSpec((nb, ds), lambda i: (i, 0)),
            pl.BlockSpec((nb, ow), lambda i: (i, 0)),
            _full_spec(wt.shape), _full_spec(wb.shape), _full_spec((1, wt.shape[1])),
            _full_spec(w1.shape), _full_spec((1, w1.shape[1])),
            _full_spec(w2.shape), _full_spec((1, w2.shape[1])),
            _full_spec(w3.shape), _full_spec((1, ncls)),
        ],
        out_specs=pl.BlockSpec((nb, ncls), lambda i: (i, 0)),
        out_shape=jax.ShapeDtypeStruct((n, ncls), jnp.float32),
    )(skip, rows, wt, wb, bd[None, :], w1, b1[None, :], w2, b2[None, :],
      w3, b3[None, :])


def kernel(features, xyz_0, xyz_1, xyz_2, xyz_3, params,
           neigh_idx_0, neigh_idx_1, neigh_idx_2, neigh_idx_3,
           sub_idx_0, sub_idx_1, sub_idx_2, sub_idx_3,
           interp_idx_0, interp_idx_1, interp_idx_2, interp_idx_3):
    p = params
    xyz = [x[0] for x in (xyz_0, xyz_1, xyz_2, xyz_3)]
    neigh = [x[0].reshape(-1).astype(jnp.int32)
             for x in (neigh_idx_0, neigh_idx_1, neigh_idx_2, neigh_idx_3)]
    sub = [x[0].reshape(-1).astype(jnp.int32)
           for x in (sub_idx_0, sub_idx_1, sub_idx_2, sub_idx_3)]
    interp = [x[0].reshape(-1).astype(jnp.int32)
              for x in (interp_idx_0, interp_idx_1, interp_idx_2, interp_idx_3)]
    feats = features[0]
    n_lvl = [x.shape[0] for x in xyz]

    d_out = [p['e%d_att1_mlp_W' % i].shape[0] for i in range(4)]
    w1_widths = [_round_up(c // 2 + 3, 16) for c in d_out]
    w2_widths = [_round_up(c // 2, 16) for c in d_out]

    feat, t1 = _fc0_mlp1(feats, xyz[0], p['fc0_W'], p['fc0_b'],
                         p['e0_mlp1_W'], p['e0_mlp1_b'], w1_widths[0])
    enc_skips = []
    dec_feat = None
    for i in range(4):
        c = d_out[i]
        d2 = c // 2
        pre = 'e%d_' % i
        n_i = n_lvl[i]
        rows1 = _gather_rows(t1, neigh[i])
        t2, fx2 = _block_a(rows1, xyz[i],
                           p[pre + 'lfa_mlp1_W'], p[pre + 'lfa_mlp1_b'],
                           p[pre + 'att1_fc_W'], p[pre + 'att1_fc_b'],
                           p[pre + 'att1_mlp_W'], p[pre + 'att1_mlp_b'],
                           p[pre + 'lfa_mlp2_W'], p[pre + 'lfa_mlp2_b'],
                           w2_widths[i], w1_widths[i])
        rows2 = _gather_rows(t2, neigh[i])
        f_enc = _block_b(rows2, fx2, feat,
                         p[pre + 'att2_fc_W'], p[pre + 'att2_fc_b'],
                         p[pre + 'att2_mlp_W'], p[pre + 'att2_mlp_b'],
                         p[pre + 'mlp2_W'], p[pre + 'mlp2_b'],
                         p[pre + 'short_W'], p[pre + 'short_b'])
        if i == 0:
            enc_skips.append(f_enc)
        prows = _gather_rows(f_enc, sub[i])
        if i < 3:
            f_s, t1 = _pool_mlp(prows, xyz[i + 1],
                                p['e%d_mlp1_W' % (i + 1)],
                                p['e%d_mlp1_b' % (i + 1)], w1_widths[i + 1])
            enc_skips.append(f_s)
            feat = f_s
        else:
            _, dec_feat = _pool_mlp(prows, None, p['dec0_W'], p['dec0_b'],
                                    p['dec0_W'].shape[1])

    for j in range(3):
        skip = enc_skips[3 - j]
        rows = _gather_rows(dec_feat, interp[3 - j])
        w = p['d%d_W' % j]
        ds = skip.shape[1]
        dec_feat = _dual_mm(skip, rows, w[:ds], w[ds:], p['d%d_b' % j])

    rows = _gather_rows(dec_feat, interp[0])
    w = p['d3_W']
    ds = enc_skips[0].shape[1]
    logits = _final(enc_skips[0], rows, w[:ds], w[ds:], p['d3_b'],
                    p['fc1_W'], p['fc1_b'], p['fc2_W'], p['fc2_b'],
                    p['fc3_W'], p['fc3_b'])
    return jnp.transpose(logits)[None]


# speedup vs baseline: 1.0475x; 1.0475x over previous
import functools

import numpy as np

import jax
import jax.numpy as jnp
from jax import lax
from jax.experimental import pallas as pl
from jax.experimental.pallas import tpu as pltpu
from jax.experimental.pallas import tpu_sc as plsc

_K = 16
_SCALE = (1.0 + 1e-06) ** -0.5
_NW = 32


def _leaky(x):
    return jnp.where(x >= 0, x, 0.2 * x)


def _mm(x, w):
    return lax.dot_general(x, w, (((x.ndim - 1,), (0,)), ((), ())),
                           preferred_element_type=jnp.float32)


def _round_up(x, m):
    return (x + m - 1) // m * m


def _largest_div(n, cap, mult=1):
    best = mult
    d = mult
    while d <= min(n, cap):
        if n % d == 0:
            best = d
        d += mult
    return best


def _blk(n, row_bytes, budget=4 * 1024 * 1024):
    p2 = n & (-n)
    cap = max(1, budget // max(row_bytes, 1))
    nb = 1
    while nb * 2 <= p2 and nb * 2 <= cap:
        nb *= 2
    return nb


def _gather_rows(table, idx):
    n_tab, d = table.shape
    ow = _round_up(d, 128)
    b = idx.shape[0]
    bp = _round_up(b, 8 * _NW)
    if bp > b:
        idx = jnp.concatenate([idx, jnp.zeros((bp - b,), jnp.int32)])
    rows_per_w = bp // _NW
    r = _largest_div(rows_per_w, cap=128, mult=8)
    n_ops = rows_per_w // r
    f_cap = max(1, min(8, (200 * 1024) // (r * d * 4)))
    f = _largest_div(n_ops, cap=f_cap)
    g_cnt = n_ops // f
    idx2 = idx.reshape(bp // r, r)

    @functools.partial(
        pl.kernel,
        mesh=plsc.VectorSubcoreMesh(core_axis_name="c", subcore_axis_name="s"),
        compiler_params=pltpu.CompilerParams(use_tc_tiling_on_sc=False),
        out_type=jax.ShapeDtypeStruct((bp, ow), jnp.float32),
        scratch_types=[
            pltpu.VMEM((n_ops, r), jnp.int32),
            pltpu.VMEM((2 * f * r, d), jnp.float32),
            pltpu.SemaphoreType.DMA,
            pltpu.SemaphoreType.DMA,
        ],
    )
    def gk(table_hbm, idx_hbm, out_hbm, idx_v, rows_v, sem, sem_wb):
        wid = lax.axis_index("s") * 2 + lax.axis_index("c")
        op0 = wid * n_ops
        pltpu.sync_copy(idx_hbm.at[pl.ds(op0, n_ops)], idx_v)

        def out_slice(g):
            if ow == d:
                return out_hbm.at[pl.ds((op0 + g * f) * r, f * r)]
            return out_hbm.at[pl.ds((op0 + g * f) * r, f * r), pl.ds(0, d)]

        def one_group(g, buf_off):
            handles = []
            for j in range(f):
                handles.append(pltpu.async_copy(
                    table_hbm.at[idx_v.at[g * f + j]],
                    rows_v.at[pl.ds(buf_off + j * r, r)], sem))
            for h in handles:
                h.wait()
            pltpu.async_copy(
                rows_v.at[pl.ds(buf_off, f * r)], out_slice(g), sem_wb)

        def drain_one(buf_off):
            pltpu.make_async_copy(
                out_slice(0), rows_v.at[pl.ds(buf_off, f * r)],
                sem_wb).wait()

        if g_cnt <= 2:
            for g in range(g_cnt):
                one_group(g, (g % 2) * f * r)
        else:
            def body(g, carry):
                buf_off = (g % 2) * f * r

                @pl.when(g >= 2)
                def _():
                    drain_one(buf_off)
                one_group(g, buf_off)
                return carry
            lax.fori_loop(0, g_cnt, body, 0)
        for t in range(min(g_cnt, 2)):
            drain_one(t * f * r)

    out = gk(table, idx2)
    return out[:b] if bp > b else out


def _full_spec(shape):
    nd = len(shape)
    return pl.BlockSpec(shape, lambda i, _nd=nd: (0,) * _nd)


def _fc0_mlp1(feats, xyz, w0, b0, w1, b1, w1_out):
    n = feats.shape[0]
    d_f = w0.shape[1]
    d2 = w1.shape[1]
    nb = _blk(n, 128 * 4 * 2)

    def body(x_ref, xyz_ref, w0_ref, b0_ref, w1_ref, b1_ref, feat_ref, t1_ref):
        x = x_ref[...]
        ft = _leaky(_SCALE * (_mm(x, w0_ref[...]) + b0_ref[...]))
        fp = _leaky(_SCALE * (_mm(ft, w1_ref[...]) + b1_ref[...]))
        feat_ref[...] = ft
        pad = w1_out - d2 - 3
        t1_ref[...] = jnp.concatenate(
            [fp, xyz_ref[...], jnp.zeros((fp.shape[0], pad), jnp.float32)],
            axis=1)

    return pl.pallas_call(
        body,
        grid=(n // nb,),
        in_specs=[
            pl.BlockSpec((nb, 3), lambda i: (i, 0)),
            pl.BlockSpec((nb, 3), lambda i: (i, 0)),
            _full_spec(w0.shape), _full_spec((1, d_f)),
            _full_spec(w1.shape), _full_spec((1, d2)),
        ],
        out_specs=[
            pl.BlockSpec((nb, d_f), lambda i: (i, 0)),
            pl.BlockSpec((nb, w1_out), lambda i: (i, 0)),
        ],
        out_shape=[
            jax.ShapeDtypeStruct((n, d_f), jnp.float32),
            jax.ShapeDtypeStruct((n, w1_out), jnp.float32),
        ],
    )(feats, xyz, w0, b0[None, :], w1, b1[None, :])


def _block_a(rows1, xyz, wl1, bl1, wfc, bfc, wm, bm, wl2, bl2, w2_out, w1):
    ow1 = rows1.shape[1]
    n = rows1.shape[0] // _K
    k = _K
    d2 = wl1.shape[1]
    c = 2 * d2

    w_rel, w_tile, w_nx = wl1[1:4], wl1[4:7], wl1[7:10]
    w_comb = jnp.zeros((w1, c), jnp.float32)
    w_comb = w_comb.at[:d2, :d2].set(jnp.eye(d2, dtype=jnp.float32))
    w_comb = w_comb.at[d2:d2 + 3, d2:].set(_SCALE * (w_nx - w_rel))
    wt_pad = jnp.zeros((3, c), jnp.float32).at[:, d2:].set(
        _SCALE * (w_rel + w_tile))
    w0_pad = jnp.zeros((1, c), jnp.float32).at[0, d2:].set(_SCALE * wl1[0])
    b_pad = jnp.zeros((1, c), jnp.float32).at[0, d2:].set(_SCALE * bl1)
    wl2_pad = jnp.zeros((c, d2), jnp.float32).at[d2:].set(wl2)

    nb = _blk(n, 3 * k * 128 * 4)

    def body(rows_ref, xyz_ref, wc_ref, wt_ref,
             w0_ref, bp_ref, wfc_ref, bfc_ref, wm_ref, bm_ref, wl2_ref,
             bl2_ref, t2_ref, fx2_ref):
        rows = rows_ref[...][:, :w1]
        xyz_b = xyz_ref[...]
        f0 = _mm(rows, wc_ref[...])
        r3 = rows.reshape(nb, k, w1)
        rel0 = xyz_b[:, None, 0:1] - r3[:, :, d2:d2 + 1]
        rel1 = xyz_b[:, None, 1:2] - r3[:, :, d2 + 1:d2 + 2]
        rel2 = xyz_b[:, None, 2:3] - r3[:, :, d2 + 2:d2 + 3]
        dis = jnp.sqrt(rel0 * rel0 + rel1 * rel1 + rel2 * rel2 + 1e-12)
        tp = _mm(xyz_b, wt_ref[...])
        f3 = (f0.reshape(nb, k, c) + tp[:, None, :]
              + dis * w0_ref[...][None] + bp_ref[...][None])
        lane = lax.broadcasted_iota(jnp.int32, (nb, k, c), 2)
        f3d = jnp.where(lane < d2, f3, _leaky(f3))
        f = f3d.reshape(nb * k, c)
        e = jnp.exp(_mm(f, wfc_ref[...]) + bfc_ref[...]).reshape(nb, k, c)
        den = jnp.sum(e, axis=1)
        num = jnp.sum(f3d * e, axis=1)
        agg = num / den
        fagg = _leaky(_SCALE * (_mm(agg, wm_ref[...]) + bm_ref[...]))
        fx2 = _leaky(_SCALE * (_mm(f, wl2_ref[...]) + bl2_ref[...]))
        pad = w2_out - d2
        if pad:
            fagg = jnp.concatenate(
                [fagg, jnp.zeros((nb, pad), jnp.float32)], axis=1)
        t2_ref[...] = fagg
        fx2_ref[...] = fx2

    return pl.pallas_call(
        body,
        grid=(n // nb,),
        in_specs=[
            pl.BlockSpec((nb * k, ow1), lambda i: (i, 0)),
            pl.BlockSpec((nb, 3), lambda i: (i, 0)),
            _full_spec((w1, c)), _full_spec((3, c)),
            _full_spec((1, c)), _full_spec((1, c)),
            _full_spec(wfc.shape), _full_spec((1, c)),
            _full_spec(wm.shape), _full_spec((1, d2)),
            _full_spec((c, d2)), _full_spec((1, d2)),
        ],
        out_specs=[
            pl.BlockSpec((nb, w2_out), lambda i: (i, 0)),
            pl.BlockSpec((nb * k, d2), lambda i: (i, 0)),
        ],
        out_shape=[
            jax.ShapeDtypeStruct((n, w2_out), jnp.float32),
            jax.ShapeDtypeStruct((n * k, d2), jnp.float32),
        ],
    )(rows1, xyz, w_comb, wt_pad, w0_pad, b_pad, wfc,
      bfc[None, :], wm, bm[None, :], wl2_pad, bl2[None, :])


def _block_b(rows2, fx2, feat, wfc, bfc, wm, bm, wmlp2, bmlp2, wsh, bsh):
    ow2 = rows2.shape[1]
    k = _K
    n = rows2.shape[0] // k
    d2 = fx2.shape[1]
    c = 2 * d2
    c2 = wmlp2.shape[1]
    d_in = feat.shape[1]
    nb = _blk(n, 3 * k * 128 * 4)

    def body(rows_ref, fx2_ref, feat_ref, wfc_ref, bfc_ref,
             wm_ref, bm_ref, wmlp2_ref, bmlp2_ref, wsh_ref, bsh_ref, out_ref):
        f_n = rows_ref[...][:, :d2]
        fxyz = fx2_ref[...]
        f = jnp.concatenate([f_n, fxyz], axis=1)
        e = jnp.exp(_mm(f, wfc_ref[...]) + bfc_ref[...]).reshape(nb, k, c)
        den = jnp.sum(e, axis=1)
        num = jnp.sum(f.reshape(nb, k, c) * e, axis=1)
        agg = num / den
        fagg = _leaky(_SCALE * (_mm(agg, wm_ref[...]) + bm_ref[...]))
        fpc = _SCALE * (_mm(fagg, wmlp2_ref[...]) + bmlp2_ref[...])
        sc = _SCALE * (_mm(feat_ref[...], wsh_ref[...]) + bsh_ref[...])
        out_ref[...] = _leaky(fpc + sc)

    return pl.pallas_call(
        body,
        grid=(n // nb,),
        in_specs=[
            pl.BlockSpec((nb * k, ow2), lambda i: (i, 0)),
            pl.BlockSpec((nb * k, d2), lambda i: (i, 0)),
            pl.BlockSpec((nb, d_in), lambda i: (i, 0)),
            _full_spec(wfc.shape), _full_spec((1, c)),
            _full_spec(wm.shape), _full_spec((1, c)),
            _full_spec(wmlp2.shape), _full_spec((1, c2)),
            _full_spec(wsh.shape), _full_spec((1, c2)),
        ],
        out_specs=pl.BlockSpec((nb, c2), lambda i: (i, 0)),
        out_shape=jax.ShapeDtypeStruct((n, c2), jnp.float32),
    )(rows2, fx2, feat, wfc, bfc[None, :], wm, bm[None, :], wmlp2,
      bmlp2[None, :], wsh, bsh[None, :])


def _pool_mlp(prows, xyz, w, b, w1_out):
    ow = prows.shape[1]
    c = w.shape[0]
    k = _K
    n = prows.shape[0] // k
    d_o = w.shape[1]
    nb = _blk(n, (k + 2) * max(c, 128) * 4)
    has_xyz = xyz is not None

    def body(*refs):
        if has_xyz:
            (rows_ref, xyz_ref, w_ref, b_ref, fs_ref, t1_ref) = refs
        else:
            (rows_ref, w_ref, b_ref, fs_ref, t1_ref) = refs
        fs = jnp.max(rows_ref[...][:, :c].reshape(nb, k, c), axis=1)
        fp = _leaky(_SCALE * (_mm(fs, w_ref[...]) + b_ref[...]))
        fs_ref[...] = fs
        if has_xyz:
            pad = w1_out - d_o - 3
            fp = jnp.concatenate(
                [fp, xyz_ref[...], jnp.zeros((nb, pad), jnp.float32)], axis=1)
        t1_ref[...] = fp

    in_specs = [pl.BlockSpec((nb * k, ow), lambda i: (i, 0))]
    args = [prows]
    if has_xyz:
        in_specs.append(pl.BlockSpec((nb, 3), lambda i: (i, 0)))
        args.append(xyz)
    in_specs += [_full_spec(w.shape), _full_spec((1, d_o))]
    args += [w, b[None, :]]

    return pl.pallas_call(
        body,
        grid=(n // nb,),
        in_specs=in_specs,
        out_specs=[
            pl.BlockSpec((nb, c), lambda i: (i, 0)),
            pl.BlockSpec((nb, w1_out), lambda i: (i, 0)),
        ],
        out_shape=[
            jax.ShapeDtypeStruct((n, c), jnp.float32),
            jax.ShapeDtypeStruct((n, w1_out), jnp.float32),
        ],
    )(*args)


def _dual_mm(skip, rows, wt, wb, b):
    n, ds = skip.shape
    ow = rows.shape[1]
    df = wb.shape[0]
    d_o = wt.shape[1]
    nb = _blk(n, (ds + ow + d_o) * 4)

    def body(s_ref, r_ref, wt_ref, wb_ref, b_ref, o_ref):
        y = _mm(s_ref[...], wt_ref[...]) + _mm(r_ref[...][:, :df], wb_ref[...])
        o_ref[...] = _leaky(_SCALE * (y + b_ref[...]))

    return pl.pallas_call(
        body,
        grid=(n // nb,),
        in_specs=[
            pl.BlockSpec((nb, ds), lambda i: (i, 0)),
            pl.BlockSpec((nb, ow), lambda i: (i, 0)),
            _full_spec(wt.shape), _full_spec(wb.shape), _full_spec((1, d_o)),
        ],
        out_specs=pl.BlockSpec((nb, d_o), lambda i: (i, 0)),
        out_shape=jax.ShapeDtypeStruct((n, d_o), jnp.float32),
    )(skip, rows, wt, wb, b[None, :])


def _final(skip, rows, wt, wb, bd, w1, b1, w2, b2, w3, b3):
    n, ds = skip.shape
    ow = rows.shape[1]
    df = wb.shape[0]
    ncls = w3.shape[1]
    nb = _blk(n, (ds + ow + 64 + 128) * 4)

    def body(s_ref, r_ref, wt_ref, wb_ref, bd_ref, w1_ref, b1_ref,
             w2_ref, b2_ref, w3_ref, b3_ref, o_ref):
        y = _mm(s_ref[...], wt_ref[...]) + _mm(r_ref[...][:, :df], wb_ref[...])
        y = _leaky(_SCALE * (y + bd_ref[...]))
        y = _leaky(_SCALE * (_mm(y, w1_ref[...]) + b1_ref[...]))
        y = _leaky(_SCALE * (_mm(y, w2_ref[...]) + b2_ref[...]))
        o_ref[...] = _mm(y, w3_ref[...]) + b3_ref[...]

    return pl.pallas_call(
        body,
        grid=(n // nb,),
        in_specs=[
            pl.BlockSpec((nb, ds), lambda i: (i, 0)),
            pl.BlockSpec((nb, ow), lambda i: (i, 0)),
            _full_spec(wt.shape), _full_spec(wb.shape), _full_spec((1, wt.shape[1])),
            _full_spec(w1.shape), _full_spec((1, w1.shape[1])),
            _full_spec(w2.shape), _full_spec((1, w2.shape[1])),
            _full_spec(w3.shape), _full_spec((1, ncls)),
        ],
        out_specs=pl.BlockSpec((nb, ncls), lambda i: (i, 0)),
        out_shape=jax.ShapeDtypeStruct((n, ncls), jnp.float32),
    )(skip, rows, wt, wb, bd[None, :], w1, b1[None, :], w2, b2[None, :],
      w3, b3[None, :])


def kernel(features, xyz_0, xyz_1, xyz_2, xyz_3, params,
           neigh_idx_0, neigh_idx_1, neigh_idx_2, neigh_idx_3,
           sub_idx_0, sub_idx_1, sub_idx_2, sub_idx_3,
           interp_idx_0, interp_idx_1, interp_idx_2, interp_idx_3):
    p = params
    xyz = [x[0] for x in (xyz_0, xyz_1, xyz_2, xyz_3)]
    neigh = [x[0].reshape(-1).astype(jnp.int32)
             for x in (neigh_idx_0, neigh_idx_1, neigh_idx_2, neigh_idx_3)]
    sub = [x[0].reshape(-1).astype(jnp.int32)
           for x in (sub_idx_0, sub_idx_1, sub_idx_2, sub_idx_3)]
    interp = [x[0].reshape(-1).astype(jnp.int32)
              for x in (interp_idx_0, interp_idx_1, interp_idx_2, interp_idx_3)]
    feats = features[0]
    n_lvl = [x.shape[0] for x in xyz]

    d_out = [p['e%d_att1_mlp_W' % i].shape[0] for i in range(4)]
    w1_widths = [_round_up(c // 2 + 3, 16) for c in d_out]
    w2_widths = [_round_up(c // 2, 16) for c in d_out]

    feat, t1 = _fc0_mlp1(feats, xyz[0], p['fc0_W'], p['fc0_b'],
                         p['e0_mlp1_W'], p['e0_mlp1_b'], w1_widths[0])
    enc_skips = []
    dec_feat = None
    for i in range(4):
        c = d_out[i]
        d2 = c // 2
        pre = 'e%d_' % i
        n_i = n_lvl[i]
        rows1 = _gather_rows(t1, neigh[i])
        t2, fx2 = _block_a(rows1, xyz[i],
                           p[pre + 'lfa_mlp1_W'], p[pre + 'lfa_mlp1_b'],
                           p[pre + 'att1_fc_W'], p[pre + 'att1_fc_b'],
                           p[pre + 'att1_mlp_W'], p[pre + 'att1_mlp_b'],
                           p[pre + 'lfa_mlp2_W'], p[pre + 'lfa_mlp2_b'],
                           w2_widths[i], w1_widths[i])
        rows2 = _gather_rows(t2, neigh[i])
        f_enc = _block_b(rows2, fx2, feat,
                         p[pre + 'att2_fc_W'], p[pre + 'att2_fc_b'],
                         p[pre + 'att2_mlp_W'], p[pre + 'att2_mlp_b'],
                         p[pre + 'mlp2_W'], p[pre + 'mlp2_b'],
                         p[pre + 'short_W'], p[pre + 'short_b'])
        if i == 0:
            enc_skips.append(f_enc)
        prows = _gather_rows(f_enc, sub[i])
        if i < 3:
            f_s, t1 = _pool_mlp(prows, xyz[i + 1],
                                p['e%d_mlp1_W' % (i + 1)],
                                p['e%d_mlp1_b' % (i + 1)], w1_widths[i + 1])
            enc_skips.append(f_s)
            feat = f_s
        else:
            _, dec_feat = _pool_mlp(prows, None, p['dec0_W'], p['dec0_b'],
                                    p['dec0_W'].shape[1])

    for j in range(3):
        skip = enc_skips[3 - j]
        rows = _gather_rows(dec_feat, interp[3 - j])
        w = p['d%d_W' % j]
        ds = skip.shape[1]
        dec_feat = _dual_mm(skip, rows, w[:ds], w[ds:], p['d%d_b' % j])

    rows = _gather_rows(dec_feat, interp[0])
    w = p['d3_W']
    ds = enc_skips[0].shape[1]
    logits = _final(enc_skips[0], rows, w[:ds], w[ds:], p['d3_b'],
                    p['fc1_W'], p['fc1_b'], p['fc2_W'], p['fc2_b'],
                    p['fc3_W'], p['fc3_b'])
    return jnp.transpose(logits)[None]


# speedup vs baseline: 1.1570x; 1.1046x over previous
import functools

import numpy as np

import jax
import jax.numpy as jnp
from jax import lax
from jax.experimental import pallas as pl
from jax.experimental.pallas import tpu as pltpu
from jax.experimental.pallas import tpu_sc as plsc

_K = 16
_SCALE = (1.0 + 1e-06) ** -0.5
_NW = 32


def _leaky(x):
    return jnp.where(x >= 0, x, 0.2 * x)


def _mm(x, w):
    return lax.dot_general(x, w, (((x.ndim - 1,), (0,)), ((), ())),
                           preferred_element_type=jnp.float32)


def _round_up(x, m):
    return (x + m - 1) // m * m


def _largest_div(n, cap, mult=1):
    best = mult
    d = mult
    while d <= min(n, cap):
        if n % d == 0:
            best = d
        d += mult
    return best


def _blk(n, row_bytes, budget=4 * 1024 * 1024):
    p2 = n & (-n)
    cap = max(1, budget // max(row_bytes, 1))
    nb = 1
    while nb * 2 <= p2 and nb * 2 <= cap:
        nb *= 2
    return nb


def _gather_rows(table, idx):
    n_tab, d = table.shape
    ow = _round_up(d, 128)
    b = idx.shape[0]
    bp = _round_up(b, 8 * _NW)
    if bp > b:
        idx = jnp.concatenate([idx, jnp.zeros((bp - b,), jnp.int32)])
    rows_per_w = bp // _NW
    r = _largest_div(rows_per_w, cap=128, mult=8)
    n_ops = rows_per_w // r
    f_cap = max(1, min(8, (200 * 1024) // (r * d * 4)))
    f = _largest_div(n_ops, cap=f_cap)
    g_cnt = n_ops // f
    idx2 = idx.reshape(bp // r, r)

    @functools.partial(
        pl.kernel,
        mesh=plsc.VectorSubcoreMesh(core_axis_name="c", subcore_axis_name="s"),
        compiler_params=pltpu.CompilerParams(use_tc_tiling_on_sc=False),
        out_type=jax.ShapeDtypeStruct((bp, ow), jnp.float32),
        scratch_types=[
            pltpu.VMEM((n_ops, r), jnp.int32),
            pltpu.VMEM((2 * f * r, d), jnp.float32),
            pltpu.SemaphoreType.DMA,
            pltpu.SemaphoreType.DMA,
        ],
    )
    def gk(table_hbm, idx_hbm, out_hbm, idx_v, rows_v, sem, sem_wb):
        wid = lax.axis_index("s") * 2 + lax.axis_index("c")
        op0 = wid * n_ops
        pltpu.sync_copy(idx_hbm.at[pl.ds(op0, n_ops)], idx_v)

        def out_slice(g):
            if ow == d:
                return out_hbm.at[pl.ds((op0 + g * f) * r, f * r)]
            return out_hbm.at[pl.ds((op0 + g * f) * r, f * r), pl.ds(0, d)]

        def one_group(g, buf_off):
            handles = []
            for j in range(f):
                handles.append(pltpu.async_copy(
                    table_hbm.at[idx_v.at[g * f + j]],
                    rows_v.at[pl.ds(buf_off + j * r, r)], sem))
            for h in handles:
                h.wait()
            pltpu.async_copy(
                rows_v.at[pl.ds(buf_off, f * r)], out_slice(g), sem_wb)

        def drain_one(buf_off):
            pltpu.make_async_copy(
                out_slice(0), rows_v.at[pl.ds(buf_off, f * r)],
                sem_wb).wait()

        if g_cnt <= 2:
            for g in range(g_cnt):
                one_group(g, (g % 2) * f * r)
        else:
            def body(g, carry):
                buf_off = (g % 2) * f * r

                @pl.when(g >= 2)
                def _():
                    drain_one(buf_off)
                one_group(g, buf_off)
                return carry
            lax.fori_loop(0, g_cnt, body, 0)
        for t in range(min(g_cnt, 2)):
            drain_one(t * f * r)

    out = gk(table, idx2)
    return out[:b] if bp > b else out


def _full_spec(shape):
    nd = len(shape)
    return pl.BlockSpec(shape, lambda i, _nd=nd: (0,) * _nd)


def _fc0_mlp1(feats, xyz, w0, b0, w1, b1, w1_out):
    n = feats.shape[0]
    d_f = w0.shape[1]
    d2 = w1.shape[1]
    nb = _blk(n, 128 * 4 * 2)

    def body(x_ref, xyz_ref, w0_ref, b0_ref, w1_ref, b1_ref, feat_ref, t1_ref):
        x = x_ref[...]
        ft = _leaky(_SCALE * (_mm(x, w0_ref[...]) + b0_ref[...]))
        fp = _leaky(_SCALE * (_mm(ft, w1_ref[...]) + b1_ref[...]))
        feat_ref[...] = ft
        pad = w1_out - d2 - 3
        t1_ref[...] = jnp.concatenate(
            [fp, xyz_ref[...], jnp.zeros((fp.shape[0], pad), jnp.float32)],
            axis=1)

    return pl.pallas_call(
        body,
        grid=(n // nb,),
        in_specs=[
            pl.BlockSpec((nb, 3), lambda i: (i, 0)),
            pl.BlockSpec((nb, 3), lambda i: (i, 0)),
            _full_spec(w0.shape), _full_spec((1, d_f)),
            _full_spec(w1.shape), _full_spec((1, d2)),
        ],
        out_specs=[
            pl.BlockSpec((nb, d_f), lambda i: (i, 0)),
            pl.BlockSpec((nb, w1_out), lambda i: (i, 0)),
        ],
        out_shape=[
            jax.ShapeDtypeStruct((n, d_f), jnp.float32),
            jax.ShapeDtypeStruct((n, w1_out), jnp.float32),
        ],
    )(feats, xyz, w0, b0[None, :], w1, b1[None, :])


def _block_a(rows1, xyz, wl1, bl1, wfc, bfc, wm, bm, wl2, bl2, w2_out, w1):
    ow1 = rows1.shape[1]
    n = rows1.shape[0] // _K
    k = _K
    d2 = wl1.shape[1]
    c = 2 * d2

    w_rel, w_tile, w_nx = wl1[1:4], wl1[4:7], wl1[7:10]
    w_comb = jnp.zeros((w1, c), jnp.float32)
    w_comb = w_comb.at[:d2, :d2].set(jnp.eye(d2, dtype=jnp.float32))
    w_comb = w_comb.at[d2:d2 + 3, d2:].set(_SCALE * (w_nx - w_rel))
    wt_pad = jnp.zeros((3, c), jnp.float32).at[:, d2:].set(
        _SCALE * (w_rel + w_tile))
    w0_pad = jnp.zeros((1, c), jnp.float32).at[0, d2:].set(_SCALE * wl1[0])
    b_pad = jnp.zeros((1, c), jnp.float32).at[0, d2:].set(_SCALE * bl1)
    wl2_pad = jnp.zeros((c, d2), jnp.float32).at[d2:].set(wl2)

    nb = _blk(n, 3 * k * 128 * 4, budget=12 * 1024 * 1024)

    def body(rows_ref, xyz_ref, wc_ref, wt_ref,
             w0_ref, bp_ref, wfc_ref, bfc_ref, wm_ref, bm_ref, wl2_ref,
             bl2_ref, t2_ref, fx2_ref):
        rows = rows_ref[...][:, :w1]
        xyz_b = xyz_ref[...]
        f0 = _mm(rows, wc_ref[...])
        r3 = rows.reshape(nb, k, w1)
        rel0 = xyz_b[:, None, 0:1] - r3[:, :, d2:d2 + 1]
        rel1 = xyz_b[:, None, 1:2] - r3[:, :, d2 + 1:d2 + 2]
        rel2 = xyz_b[:, None, 2:3] - r3[:, :, d2 + 2:d2 + 3]
        dis = jnp.sqrt(rel0 * rel0 + rel1 * rel1 + rel2 * rel2 + 1e-12)
        tp = _mm(xyz_b, wt_ref[...])
        f3 = (f0.reshape(nb, k, c) + tp[:, None, :]
              + dis * w0_ref[...][None] + bp_ref[...][None])
        lane = lax.broadcasted_iota(jnp.int32, (nb, k, c), 2)
        f3d = jnp.where(lane < d2, f3, _leaky(f3))
        f = f3d.reshape(nb * k, c)
        e = jnp.exp(_mm(f, wfc_ref[...]) + bfc_ref[...]).reshape(nb, k, c)
        den = jnp.sum(e, axis=1)
        num = jnp.sum(f3d * e, axis=1)
        agg = num / den
        fagg = _leaky(_SCALE * (_mm(agg, wm_ref[...]) + bm_ref[...]))
        fx2 = _leaky(_SCALE * (_mm(f, wl2_ref[...]) + bl2_ref[...]))
        pad = w2_out - d2
        if pad:
            fagg = jnp.concatenate(
                [fagg, jnp.zeros((nb, pad), jnp.float32)], axis=1)
        t2_ref[...] = fagg
        fx2_ref[...] = fx2

    return pl.pallas_call(
        body,
        grid=(n // nb,),
        in_specs=[
            pl.BlockSpec((nb * k, ow1), lambda i: (i, 0)),
            pl.BlockSpec((nb, 3), lambda i: (i, 0)),
            _full_spec((w1, c)), _full_spec((3, c)),
            _full_spec((1, c)), _full_spec((1, c)),
            _full_spec(wfc.shape), _full_spec((1, c)),
            _full_spec(wm.shape), _full_spec((1, d2)),
            _full_spec((c, d2)), _full_spec((1, d2)),
        ],
        out_specs=[
            pl.BlockSpec((nb, w2_out), lambda i: (i, 0)),
            pl.BlockSpec((nb * k, d2), lambda i: (i, 0)),
        ],
        out_shape=[
            jax.ShapeDtypeStruct((n, w2_out), jnp.float32),
            jax.ShapeDtypeStruct((n * k, d2), jnp.float32),
        ],
    )(rows1, xyz, w_comb, wt_pad, w0_pad, b_pad, wfc,
      bfc[None, :], wm, bm[None, :], wl2_pad, bl2[None, :])


def _block_b(rows2, fx2, feat, wfc, bfc, wm, bm, wmlp2, bmlp2, wsh, bsh):
    ow2 = rows2.shape[1]
    k = _K
    n = rows2.shape[0] // k
    d2 = fx2.shape[1]
    c = 2 * d2
    c2 = wmlp2.shape[1]
    d_in = feat.shape[1]
    nb = _blk(n, 3 * k * 128 * 4, budget=12 * 1024 * 1024)

    def body(rows_ref, fx2_ref, feat_ref, wfc_ref, bfc_ref,
             wm_ref, bm_ref, wmlp2_ref, bmlp2_ref, wsh_ref, bsh_ref, out_ref):
        f_n = rows_ref[...][:, :d2]
        fxyz = fx2_ref[...]
        f = jnp.concatenate([f_n, fxyz], axis=1)
        e = jnp.exp(_mm(f, wfc_ref[...]) + bfc_ref[...]).reshape(nb, k, c)
        den = jnp.sum(e, axis=1)
        num = jnp.sum(f.reshape(nb, k, c) * e, axis=1)
        agg = num / den
        fagg = _leaky(_SCALE * (_mm(agg, wm_ref[...]) + bm_ref[...]))
        fpc = _SCALE * (_mm(fagg, wmlp2_ref[...]) + bmlp2_ref[...])
        sc = _SCALE * (_mm(feat_ref[...], wsh_ref[...]) + bsh_ref[...])
        out_ref[...] = _leaky(fpc + sc)

    return pl.pallas_call(
        body,
        grid=(n // nb,),
        in_specs=[
            pl.BlockSpec((nb * k, ow2), lambda i: (i, 0)),
            pl.BlockSpec((nb * k, d2), lambda i: (i, 0)),
            pl.BlockSpec((nb, d_in), lambda i: (i, 0)),
            _full_spec(wfc.shape), _full_spec((1, c)),
            _full_spec(wm.shape), _full_spec((1, c)),
            _full_spec(wmlp2.shape), _full_spec((1, c2)),
            _full_spec(wsh.shape), _full_spec((1, c2)),
        ],
        out_specs=pl.BlockSpec((nb, c2), lambda i: (i, 0)),
        out_shape=jax.ShapeDtypeStruct((n, c2), jnp.float32),
    )(rows2, fx2, feat, wfc, bfc[None, :], wm, bm[None, :], wmlp2,
      bmlp2[None, :], wsh, bsh[None, :])


def _pool_mlp(prows, xyz, w, b, w1_out):
    ow = prows.shape[1]
    c = w.shape[0]
    k = _K
    n = prows.shape[0] // k
    d_o = w.shape[1]
    nb = _blk(n, (k + 2) * max(c, 128) * 4)
    has_xyz = xyz is not None

    def body(*refs):
        if has_xyz:
            (rows_ref, xyz_ref, w_ref, b_ref, fs_ref, t1_ref) = refs
        else:
            (rows_ref, w_ref, b_ref, fs_ref, t1_ref) = refs
        fs = jnp.max(rows_ref[...][:, :c].reshape(nb, k, c), axis=1)
        fp = _leaky(_SCALE * (_mm(fs, w_ref[...]) + b_ref[...]))
        fs_ref[...] = fs
        if has_xyz:
            pad = w1_out - d_o - 3
            fp = jnp.concatenate(
                [fp, xyz_ref[...], jnp.zeros((nb, pad), jnp.float32)], axis=1)
        t1_ref[...] = fp

    in_specs = [pl.BlockSpec((nb * k, ow), lambda i: (i, 0))]
    args = [prows]
    if has_xyz:
        in_specs.append(pl.BlockSpec((nb, 3), lambda i: (i, 0)))
        args.append(xyz)
    in_specs += [_full_spec(w.shape), _full_spec((1, d_o))]
    args += [w, b[None, :]]

    return pl.pallas_call(
        body,
        grid=(n // nb,),
        in_specs=in_specs,
        out_specs=[
            pl.BlockSpec((nb, c), lambda i: (i, 0)),
            pl.BlockSpec((nb, w1_out), lambda i: (i, 0)),
        ],
        out_shape=[
            jax.ShapeDtypeStruct((n, c), jnp.float32),
            jax.ShapeDtypeStruct((n, w1_out), jnp.float32),
        ],
    )(*args)


def _dual_mm(skip, rows, wt, wb, b):
    n, ds = skip.shape
    ow = rows.shape[1]
    df = wb.shape[0]
    d_o = wt.shape[1]
    nb = _blk(n, (ds + ow + d_o) * 4)

    def body(s_ref, r_ref, wt_ref, wb_ref, b_ref, o_ref):
        y = _mm(s_ref[...], wt_ref[...]) + _mm(r_ref[...][:, :df], wb_ref[...])
        o_ref[...] = _leaky(_SCALE * (y + b_ref[...]))

    return pl.pallas_call(
        body,
        grid=(n // nb,),
        in_specs=[
            pl.BlockSpec((nb, ds), lambda i: (i, 0)),
            pl.BlockSpec((nb, ow), lambda i: (i, 0)),
            _full_spec(wt.shape), _full_spec(wb.shape), _full_spec((1, d_o)),
        ],
        out_specs=pl.BlockSpec((nb, d_o), lambda i: (i, 0)),
        out_shape=jax.ShapeDtypeStruct((n, d_o), jnp.float32),
    )(skip, rows, wt, wb, b[None, :])


def _final(skip, rows, wt, wb, bd, w1, b1, w2, b2, w3, b3):
    n, ds = skip.shape
    ow = rows.shape[1]
    df = wb.shape[0]
    ncls = w3.shape[1]
    nb = _blk(n, (ds + ow + 64 + 128) * 4)

    def body(s_ref, r_ref, wt_ref, wb_ref, bd_ref, w1_ref, b1_ref,
             w2_ref, b2_ref, w3_ref, b3_ref, o_ref):
        y = _mm(s_ref[...], wt_ref[...]) + _mm(r_ref[...][:, :df], wb_ref[...])
        y = _leaky(_SCALE * (y + bd_ref[...]))
        y = _leaky(_SCALE * (_mm(y, w1_ref[...]) + b1_ref[...]))
        y = _leaky(_SCALE * (_mm(y, w2_ref[...]) + b2_ref[...]))
        o_ref[...] = _mm(y, w3_ref[...]) + b3_ref[...]

    return pl.pallas_call(
        body,
        grid=(n // nb,),
        in_specs=[
            pl.BlockSpec((nb, ds), lambda i: (i, 0)),
            pl.BlockSpec((nb, ow), lambda i: (i, 0)),
            _full_spec(wt.shape), _full_spec(wb.shape), _full_spec((1, wt.shape[1])),
            _full_spec(w1.shape), _full_spec((1, w1.shape[1])),
            _full_spec(w2.shape), _full_spec((1, w2.shape[1])),
            _full_spec(w3.shape), _full_spec((1, ncls)),
        ],
        out_specs=pl.BlockSpec((nb, ncls), lambda i: (i, 0)),
        out_shape=jax.ShapeDtypeStruct((n, ncls), jnp.float32),
    )(skip, rows, wt, wb, bd[None, :], w1, b1[None, :], w2, b2[None, :],
      w3, b3[None, :])


def kernel(features, xyz_0, xyz_1, xyz_2, xyz_3, params,
           neigh_idx_0, neigh_idx_1, neigh_idx_2, neigh_idx_3,
           sub_idx_0, sub_idx_1, sub_idx_2, sub_idx_3,
           interp_idx_0, interp_idx_1, interp_idx_2, interp_idx_3):
    p = params
    xyz = [x[0] for x in (xyz_0, xyz_1, xyz_2, xyz_3)]
    neigh = [x[0].reshape(-1).astype(jnp.int32)
             for x in (neigh_idx_0, neigh_idx_1, neigh_idx_2, neigh_idx_3)]
    sub = [x[0].reshape(-1).astype(jnp.int32)
           for x in (sub_idx_0, sub_idx_1, sub_idx_2, sub_idx_3)]
    interp = [x[0].reshape(-1).astype(jnp.int32)
              for x in (interp_idx_0, interp_idx_1, interp_idx_2, interp_idx_3)]
    feats = features[0]
    n_lvl = [x.shape[0] for x in xyz]

    d_out = [p['e%d_att1_mlp_W' % i].shape[0] for i in range(4)]
    w1_widths = [_round_up(c // 2 + 3, 16) for c in d_out]
    w2_widths = [_round_up(c // 2, 16) for c in d_out]

    feat, t1 = _fc0_mlp1(feats, xyz[0], p['fc0_W'], p['fc0_b'],
                         p['e0_mlp1_W'], p['e0_mlp1_b'], w1_widths[0])
    enc_skips = []
    dec_feat = None
    for i in range(4):
        c = d_out[i]
        d2 = c // 2
        pre = 'e%d_' % i
        n_i = n_lvl[i]
        rows1 = _gather_rows(t1, neigh[i])
        t2, fx2 = _block_a(rows1, xyz[i],
                           p[pre + 'lfa_mlp1_W'], p[pre + 'lfa_mlp1_b'],
                           p[pre + 'att1_fc_W'], p[pre + 'att1_fc_b'],
                           p[pre + 'att1_mlp_W'], p[pre + 'att1_mlp_b'],
                           p[pre + 'lfa_mlp2_W'], p[pre + 'lfa_mlp2_b'],
                           w2_widths[i], w1_widths[i])
        rows2 = _gather_rows(t2, neigh[i])
        f_enc = _block_b(rows2, fx2, feat,
                         p[pre + 'att2_fc_W'], p[pre + 'att2_fc_b'],
                         p[pre + 'att2_mlp_W'], p[pre + 'att2_mlp_b'],
                         p[pre + 'mlp2_W'], p[pre + 'mlp2_b'],
                         p[pre + 'short_W'], p[pre + 'short_b'])
        if i == 0:
            enc_skips.append(f_enc)
        prows = _gather_rows(f_enc, sub[i])
        if i < 3:
            f_s, t1 = _pool_mlp(prows, xyz[i + 1],
                                p['e%d_mlp1_W' % (i + 1)],
                                p['e%d_mlp1_b' % (i + 1)], w1_widths[i + 1])
            enc_skips.append(f_s)
            feat = f_s
        else:
            _, dec_feat = _pool_mlp(prows, None, p['dec0_W'], p['dec0_b'],
                                    p['dec0_W'].shape[1])

    for j in range(3):
        skip = enc_skips[3 - j]
        rows = _gather_rows(dec_feat, interp[3 - j])
        w = p['d%d_W' % j]
        ds = skip.shape[1]
        dec_feat = _dual_mm(skip, rows, w[:ds], w[ds:], p['d%d_b' % j])

    rows = _gather_rows(dec_feat, interp[0])
    w = p['d3_W']
    ds = enc_skips[0].shape[1]
    logits = _final(enc_skips[0], rows, w[:ds], w[ds:], p['d3_b'],
                    p['fc1_W'], p['fc1_b'], p['fc2_W'], p['fc2_b'],
                    p['fc3_W'], p['fc3_b'])
    return jnp.transpose(logits)[None]


# speedup vs baseline: 1.1696x; 1.0109x over previous
import functools

import numpy as np

import jax
import jax.numpy as jnp
from jax import lax
from jax.experimental import pallas as pl
from jax.experimental.pallas import tpu as pltpu
from jax.experimental.pallas import tpu_sc as plsc

_K = 16
_SCALE = (1.0 + 1e-06) ** -0.5
_NW = 32


def _leaky(x):
    return jnp.where(x >= 0, x, 0.2 * x)


def _mm(x, w):
    return lax.dot_general(x, w, (((x.ndim - 1,), (0,)), ((), ())),
                           preferred_element_type=jnp.float32)


def _round_up(x, m):
    return (x + m - 1) // m * m


def _largest_div(n, cap, mult=1):
    best = mult
    d = mult
    while d <= min(n, cap):
        if n % d == 0:
            best = d
        d += mult
    return best


def _blk(n, row_bytes, budget=4 * 1024 * 1024):
    p2 = n & (-n)
    cap = max(1, budget // max(row_bytes, 1))
    nb = 1
    while nb * 2 <= p2 and nb * 2 <= cap:
        nb *= 2
    return nb


def _gather_rows(table, idx):
    n_tab, d = table.shape
    ow = _round_up(d, 128)
    b = idx.shape[0]
    bp = _round_up(b, 8 * _NW)
    if bp > b:
        idx = jnp.concatenate([idx, jnp.zeros((bp - b,), jnp.int32)])
    rows_per_w = bp // _NW
    r = _largest_div(rows_per_w, cap=128, mult=8)
    n_ops = rows_per_w // r
    f_cap = max(1, min(8, (200 * 1024) // (r * d * 4)))
    f = _largest_div(n_ops, cap=f_cap)
    g_cnt = n_ops // f
    idx2 = idx.reshape(bp // r, r)

    @functools.partial(
        pl.kernel,
        mesh=plsc.VectorSubcoreMesh(core_axis_name="c", subcore_axis_name="s"),
        compiler_params=pltpu.CompilerParams(use_tc_tiling_on_sc=False),
        out_type=jax.ShapeDtypeStruct((bp, ow), jnp.float32),
        scratch_types=[
            pltpu.VMEM((n_ops, r), jnp.int32),
            pltpu.VMEM((2 * f * r, d), jnp.float32),
            pltpu.SemaphoreType.DMA,
            pltpu.SemaphoreType.DMA,
        ],
    )
    def gk(table_hbm, idx_hbm, out_hbm, idx_v, rows_v, sem, sem_wb):
        wid = lax.axis_index("s") * 2 + lax.axis_index("c")
        op0 = wid * n_ops
        pltpu.sync_copy(idx_hbm.at[pl.ds(op0, n_ops)], idx_v)

        def out_slice(g):
            if ow == d:
                return out_hbm.at[pl.ds((op0 + g * f) * r, f * r)]
            return out_hbm.at[pl.ds((op0 + g * f) * r, f * r), pl.ds(0, d)]

        def one_group(g, buf_off):
            handles = []
            for j in range(f):
                handles.append(pltpu.async_copy(
                    table_hbm.at[idx_v.at[g * f + j]],
                    rows_v.at[pl.ds(buf_off + j * r, r)], sem))
            for h in handles:
                h.wait()
            pltpu.async_copy(
                rows_v.at[pl.ds(buf_off, f * r)], out_slice(g), sem_wb)

        def drain_one(buf_off):
            pltpu.make_async_copy(
                out_slice(0), rows_v.at[pl.ds(buf_off, f * r)],
                sem_wb).wait()

        if g_cnt <= 2:
            for g in range(g_cnt):
                one_group(g, (g % 2) * f * r)
        else:
            def body(g, carry):
                buf_off = (g % 2) * f * r

                @pl.when(g >= 2)
                def _():
                    drain_one(buf_off)
                one_group(g, buf_off)
                return carry
            lax.fori_loop(0, g_cnt, body, 0)
        for t in range(min(g_cnt, 2)):
            drain_one(t * f * r)

    out = gk(table, idx2)
    return out[:b] if bp > b else out


def _full_spec(shape):
    nd = len(shape)
    return pl.BlockSpec(shape, lambda i, _nd=nd: (0,) * _nd)


def _fc0_mlp1(feats, xyz, w0, b0, w1, b1, w1_out):
    n = feats.shape[0]
    d_f = w0.shape[1]
    d2 = w1.shape[1]
    nb = _blk(n, 128 * 4 * 2)

    def body(x_ref, xyz_ref, w0_ref, b0_ref, w1_ref, b1_ref, feat_ref, t1_ref):
        x = x_ref[...]
        ft = _leaky(_SCALE * (_mm(x, w0_ref[...]) + b0_ref[...]))
        fp = _leaky(_SCALE * (_mm(ft, w1_ref[...]) + b1_ref[...]))
        feat_ref[...] = ft
        pad = w1_out - d2 - 3
        t1_ref[...] = jnp.concatenate(
            [fp, xyz_ref[...], jnp.zeros((fp.shape[0], pad), jnp.float32)],
            axis=1)

    return pl.pallas_call(
        body,
        grid=(n // nb,),
        in_specs=[
            pl.BlockSpec((nb, 3), lambda i: (i, 0)),
            pl.BlockSpec((nb, 3), lambda i: (i, 0)),
            _full_spec(w0.shape), _full_spec((1, d_f)),
            _full_spec(w1.shape), _full_spec((1, d2)),
        ],
        out_specs=[
            pl.BlockSpec((nb, d_f), lambda i: (i, 0)),
            pl.BlockSpec((nb, w1_out), lambda i: (i, 0)),
        ],
        out_shape=[
            jax.ShapeDtypeStruct((n, d_f), jnp.float32),
            jax.ShapeDtypeStruct((n, w1_out), jnp.float32),
        ],
    )(feats, xyz, w0, b0[None, :], w1, b1[None, :])


def _block_a(rows1, xyz, wl1, bl1, wfc, bfc, wm, bm, wl2, bl2, w2_out, w1):
    ow1 = rows1.shape[1]
    n = rows1.shape[0] // _K
    k = _K
    d2 = wl1.shape[1]
    c = 2 * d2

    w_rel, w_tile, w_nx = wl1[1:4], wl1[4:7], wl1[7:10]
    w_comb = jnp.zeros((w1, c), jnp.float32)
    w_comb = w_comb.at[:d2, :d2].set(jnp.eye(d2, dtype=jnp.float32))
    w_comb = w_comb.at[d2:d2 + 3, d2:].set(_SCALE * (w_nx - w_rel))
    wt_pad = jnp.zeros((3, c), jnp.float32).at[:, d2:].set(
        _SCALE * (w_rel + w_tile))
    w0_pad = jnp.zeros((1, c), jnp.float32).at[0, d2:].set(_SCALE * wl1[0])
    b_pad = jnp.zeros((1, c), jnp.float32).at[0, d2:].set(_SCALE * bl1)
    wl2_pad = jnp.zeros((c, d2), jnp.float32).at[d2:].set(wl2)

    nb = _blk(n, 3 * k * 128 * 4, budget=24 * 1024 * 1024)

    def body(rows_ref, xyz_ref, wc_ref, wt_ref,
             w0_ref, bp_ref, wfc_ref, bfc_ref, wm_ref, bm_ref, wl2_ref,
             bl2_ref, t2_ref, fx2_ref):
        rows = rows_ref[...][:, :w1]
        xyz_b = xyz_ref[...]
        f0 = _mm(rows, wc_ref[...])
        r3 = rows.reshape(nb, k, w1)
        rel0 = xyz_b[:, None, 0:1] - r3[:, :, d2:d2 + 1]
        rel1 = xyz_b[:, None, 1:2] - r3[:, :, d2 + 1:d2 + 2]
        rel2 = xyz_b[:, None, 2:3] - r3[:, :, d2 + 2:d2 + 3]
        dis = jnp.sqrt(rel0 * rel0 + rel1 * rel1 + rel2 * rel2 + 1e-12)
        tp = _mm(xyz_b, wt_ref[...])
        f3 = (f0.reshape(nb, k, c) + tp[:, None, :]
              + dis * w0_ref[...][None] + bp_ref[...][None])
        lane = lax.broadcasted_iota(jnp.int32, (nb, k, c), 2)
        f3d = jnp.where(lane < d2, f3, _leaky(f3))
        f = f3d.reshape(nb * k, c)
        e = jnp.exp(_mm(f, wfc_ref[...]) + bfc_ref[...]).reshape(nb, k, c)
        den = jnp.sum(e, axis=1)
        num = jnp.sum(f3d * e, axis=1)
        agg = num / den
        fagg = _leaky(_SCALE * (_mm(agg, wm_ref[...]) + bm_ref[...]))
        fx2 = _leaky(_SCALE * (_mm(f, wl2_ref[...]) + bl2_ref[...]))
        pad = w2_out - d2
        if pad:
            fagg = jnp.concatenate(
                [fagg, jnp.zeros((nb, pad), jnp.float32)], axis=1)
        t2_ref[...] = fagg
        fx2_ref[...] = fx2

    return pl.pallas_call(
        body,
        grid=(n // nb,),
        in_specs=[
            pl.BlockSpec((nb * k, ow1), lambda i: (i, 0)),
            pl.BlockSpec((nb, 3), lambda i: (i, 0)),
            _full_spec((w1, c)), _full_spec((3, c)),
            _full_spec((1, c)), _full_spec((1, c)),
            _full_spec(wfc.shape), _full_spec((1, c)),
            _full_spec(wm.shape), _full_spec((1, d2)),
            _full_spec((c, d2)), _full_spec((1, d2)),
        ],
        out_specs=[
            pl.BlockSpec((nb, w2_out), lambda i: (i, 0)),
            pl.BlockSpec((nb * k, d2), lambda i: (i, 0)),
        ],
        out_shape=[
            jax.ShapeDtypeStruct((n, w2_out), jnp.float32),
            jax.ShapeDtypeStruct((n * k, d2), jnp.float32),
        ],
    )(rows1, xyz, w_comb, wt_pad, w0_pad, b_pad, wfc,
      bfc[None, :], wm, bm[None, :], wl2_pad, bl2[None, :])


def _block_b(rows2, fx2, feat, wfc, bfc, wm, bm, wmlp2, bmlp2, wsh, bsh):
    ow2 = rows2.shape[1]
    k = _K
    n = rows2.shape[0] // k
    d2 = fx2.shape[1]
    c = 2 * d2
    c2 = wmlp2.shape[1]
    d_in = feat.shape[1]
    nb = _blk(n, 3 * k * 128 * 4, budget=24 * 1024 * 1024)

    def body(rows_ref, fx2_ref, feat_ref, wfc_ref, bfc_ref,
             wm_ref, bm_ref, wmlp2_ref, bmlp2_ref, wsh_ref, bsh_ref, out_ref):
        f_n = rows_ref[...][:, :d2]
        fxyz = fx2_ref[...]
        f = jnp.concatenate([f_n, fxyz], axis=1)
        e = jnp.exp(_mm(f, wfc_ref[...]) + bfc_ref[...]).reshape(nb, k, c)
        den = jnp.sum(e, axis=1)
        num = jnp.sum(f.reshape(nb, k, c) * e, axis=1)
        agg = num / den
        fagg = _leaky(_SCALE * (_mm(agg, wm_ref[...]) + bm_ref[...]))
        fpc = _SCALE * (_mm(fagg, wmlp2_ref[...]) + bmlp2_ref[...])
        sc = _SCALE * (_mm(feat_ref[...], wsh_ref[...]) + bsh_ref[...])
        out_ref[...] = _leaky(fpc + sc)

    return pl.pallas_call(
        body,
        grid=(n // nb,),
        in_specs=[
            pl.BlockSpec((nb * k, ow2), lambda i: (i, 0)),
            pl.BlockSpec((nb * k, d2), lambda i: (i, 0)),
            pl.BlockSpec((nb, d_in), lambda i: (i, 0)),
            _full_spec(wfc.shape), _full_spec((1, c)),
            _full_spec(wm.shape), _full_spec((1, c)),
            _full_spec(wmlp2.shape), _full_spec((1, c2)),
            _full_spec(wsh.shape), _full_spec((1, c2)),
        ],
        out_specs=pl.BlockSpec((nb, c2), lambda i: (i, 0)),
        out_shape=jax.ShapeDtypeStruct((n, c2), jnp.float32),
    )(rows2, fx2, feat, wfc, bfc[None, :], wm, bm[None, :], wmlp2,
      bmlp2[None, :], wsh, bsh[None, :])


def _pool_mlp(prows, xyz, w, b, w1_out):
    ow = prows.shape[1]
    c = w.shape[0]
    k = _K
    n = prows.shape[0] // k
    d_o = w.shape[1]
    nb = _blk(n, (k + 2) * max(c, 128) * 4, budget=12 * 1024 * 1024)
    has_xyz = xyz is not None

    def body(*refs):
        if has_xyz:
            (rows_ref, xyz_ref, w_ref, b_ref, fs_ref, t1_ref) = refs
        else:
            (rows_ref, w_ref, b_ref, fs_ref, t1_ref) = refs
        fs = jnp.max(rows_ref[...][:, :c].reshape(nb, k, c), axis=1)
        fp = _leaky(_SCALE * (_mm(fs, w_ref[...]) + b_ref[...]))
        fs_ref[...] = fs
        if has_xyz:
            pad = w1_out - d_o - 3
            fp = jnp.concatenate(
                [fp, xyz_ref[...], jnp.zeros((nb, pad), jnp.float32)], axis=1)
        t1_ref[...] = fp

    in_specs = [pl.BlockSpec((nb * k, ow), lambda i: (i, 0))]
    args = [prows]
    if has_xyz:
        in_specs.append(pl.BlockSpec((nb, 3), lambda i: (i, 0)))
        args.append(xyz)
    in_specs += [_full_spec(w.shape), _full_spec((1, d_o))]
    args += [w, b[None, :]]

    return pl.pallas_call(
        body,
        grid=(n // nb,),
        in_specs=in_specs,
        out_specs=[
            pl.BlockSpec((nb, c), lambda i: (i, 0)),
            pl.BlockSpec((nb, w1_out), lambda i: (i, 0)),
        ],
        out_shape=[
            jax.ShapeDtypeStruct((n, c), jnp.float32),
            jax.ShapeDtypeStruct((n, w1_out), jnp.float32),
        ],
    )(*args)


def _dual_mm(skip, rows, wt, wb, b):
    n, ds = skip.shape
    ow = rows.shape[1]
    df = wb.shape[0]
    d_o = wt.shape[1]
    nb = _blk(n, (ds + ow + d_o) * 4)

    def body(s_ref, r_ref, wt_ref, wb_ref, b_ref, o_ref):
        y = _mm(s_ref[...], wt_ref[...]) + _mm(r_ref[...][:, :df], wb_ref[...])
        o_ref[...] = _leaky(_SCALE * (y + b_ref[...]))

    return pl.pallas_call(
        body,
        grid=(n // nb,),
        in_specs=[
            pl.BlockSpec((nb, ds), lambda i: (i, 0)),
            pl.BlockSpec((nb, ow), lambda i: (i, 0)),
            _full_spec(wt.shape), _full_spec(wb.shape), _full_spec((1, d_o)),
        ],
        out_specs=pl.BlockSpec((nb, d_o), lambda i: (i, 0)),
        out_shape=jax.ShapeDtypeStruct((n, d_o), jnp.float32),
    )(skip, rows, wt, wb, b[None, :])


def _final(skip, rows, wt, wb, bd, w1, b1, w2, b2, w3, b3):
    n, ds = skip.shape
    ow = rows.shape[1]
    df = wb.shape[0]
    ncls = w3.shape[1]
    nb = _blk(n, (ds + ow + 64 + 128) * 4)

    def body(s_ref, r_ref, wt_ref, wb_ref, bd_ref, w1_ref, b1_ref,
             w2_ref, b2_ref, w3_ref, b3_ref, o_ref):
        y = _mm(s_ref[...], wt_ref[...]) + _mm(r_ref[...][:, :df], wb_ref[...])
        y = _leaky(_SCALE * (y + bd_ref[...]))
        y = _leaky(_SCALE * (_mm(y, w1_ref[...]) + b1_ref[...]))
        y = _leaky(_SCALE * (_mm(y, w2_ref[...]) + b2_ref[...]))
        o_ref[...] = _mm(y, w3_ref[...]) + b3_ref[...]

    return pl.pallas_call(
        body,
        grid=(n // nb,),
        in_specs=[
            pl.BlockSpec((nb, ds), lambda i: (i, 0)),
            pl.BlockSpec((nb, ow), lambda i: (i, 0)),
            _full_spec(wt.shape), _full_spec(wb.shape), _full_spec((1, wt.shape[1])),
            _full_spec(w1.shape), _full_spec((1, w1.shape[1])),
            _full_spec(w2.shape), _full_spec((1, w2.shape[1])),
            _full_spec(w3.shape), _full_spec((1, ncls)),
        ],
        out_specs=pl.BlockSpec((nb, ncls), lambda i: (i, 0)),
        out_shape=jax.ShapeDtypeStruct((n, ncls), jnp.float32),
    )(skip, rows, wt, wb, bd[None, :], w1, b1[None, :], w2, b2[None, :],
      w3, b3[None, :])


def kernel(features, xyz_0, xyz_1, xyz_2, xyz_3, params,
           neigh_idx_0, neigh_idx_1, neigh_idx_2, neigh_idx_3,
           sub_idx_0, sub_idx_1, sub_idx_2, sub_idx_3,
           interp_idx_0, interp_idx_1, interp_idx_2, interp_idx_3):
    p = params
    xyz = [x[0] for x in (xyz_0, xyz_1, xyz_2, xyz_3)]
    neigh = [x[0].reshape(-1).astype(jnp.int32)
             for x in (neigh_idx_0, neigh_idx_1, neigh_idx_2, neigh_idx_3)]
    sub = [x[0].reshape(-1).astype(jnp.int32)
           for x in (sub_idx_0, sub_idx_1, sub_idx_2, sub_idx_3)]
    interp = [x[0].reshape(-1).astype(jnp.int32)
              for x in (interp_idx_0, interp_idx_1, interp_idx_2, interp_idx_3)]
    feats = features[0]
    n_lvl = [x.shape[0] for x in xyz]

    d_out = [p['e%d_att1_mlp_W' % i].shape[0] for i in range(4)]
    w1_widths = [_round_up(c // 2 + 3, 16) for c in d_out]
    w2_widths = [_round_up(c // 2, 16) for c in d_out]

    feat, t1 = _fc0_mlp1(feats, xyz[0], p['fc0_W'], p['fc0_b'],
                         p['e0_mlp1_W'], p['e0_mlp1_b'], w1_widths[0])
    enc_skips = []
    dec_feat = None
    for i in range(4):
        c = d_out[i]
        d2 = c // 2
        pre = 'e%d_' % i
        n_i = n_lvl[i]
        rows1 = _gather_rows(t1, neigh[i])
        t2, fx2 = _block_a(rows1, xyz[i],
                           p[pre + 'lfa_mlp1_W'], p[pre + 'lfa_mlp1_b'],
                           p[pre + 'att1_fc_W'], p[pre + 'att1_fc_b'],
                           p[pre + 'att1_mlp_W'], p[pre + 'att1_mlp_b'],
                           p[pre + 'lfa_mlp2_W'], p[pre + 'lfa_mlp2_b'],
                           w2_widths[i], w1_widths[i])
        rows2 = _gather_rows(t2, neigh[i])
        f_enc = _block_b(rows2, fx2, feat,
                         p[pre + 'att2_fc_W'], p[pre + 'att2_fc_b'],
                         p[pre + 'att2_mlp_W'], p[pre + 'att2_mlp_b'],
                         p[pre + 'mlp2_W'], p[pre + 'mlp2_b'],
                         p[pre + 'short_W'], p[pre + 'short_b'])
        if i == 0:
            enc_skips.append(f_enc)
        prows = _gather_rows(f_enc, sub[i])
        if i < 3:
            f_s, t1 = _pool_mlp(prows, xyz[i + 1],
                                p['e%d_mlp1_W' % (i + 1)],
                                p['e%d_mlp1_b' % (i + 1)], w1_widths[i + 1])
            enc_skips.append(f_s)
            feat = f_s
        else:
            _, dec_feat = _pool_mlp(prows, None, p['dec0_W'], p['dec0_b'],
                                    p['dec0_W'].shape[1])

    for j in range(3):
        skip = enc_skips[3 - j]
        rows = _gather_rows(dec_feat, interp[3 - j])
        w = p['d%d_W' % j]
        ds = skip.shape[1]
        dec_feat = _dual_mm(skip, rows, w[:ds], w[ds:], p['d%d_b' % j])

    rows = _gather_rows(dec_feat, interp[0])
    w = p['d3_W']
    ds = enc_skips[0].shape[1]
    logits = _final(enc_skips[0], rows, w[:ds], w[ds:], p['d3_b'],
                    p['fc1_W'], p['fc1_b'], p['fc2_W'], p['fc2_b'],
                    p['fc3_W'], p['fc3_b'])
    return jnp.transpose(logits)[None]
